# baseline (speedup 1.0000x reference)
import jax
import jax.numpy as jnp
from jax import lax
from jax.experimental import pallas as pl
from jax.experimental.pallas import tpu as pltpu
from jax.experimental.pallas import tpu_sc as plsc

_N = 50000
_D = 256
_S = 512
_NC = 2
_NS = 16
_NW = _NC * _NS
_R = 112
_L = 16
_SEG_PER_TILE = _S // _NW
_SP = 528
_BS = 2000
_G = _N // _BS


def _tc_fn(xb_ref, seg_ref, w_ref, b_ref, st_ref, wgt_ref):
    i = pl.program_id(0)

    @pl.when(i == 0)
    def _init():
        st_ref[...] = jnp.zeros_like(st_ref)

    seg = seg_ref[0]
    lanes = lax.broadcasted_iota(jnp.int32, (_BS, _SP), 1)
    st_ref[...] += jnp.sum((seg < lanes).astype(jnp.int32), axis=0,
                           keepdims=True)
    xb = xb_ref[...]
    z = jnp.sum(xb * w_ref[...], axis=1, keepdims=True) + b_ref[0, 0]
    wgt_ref[...] = jax.nn.sigmoid(z).reshape(1, _BS, 1)


def _sc_body(x_hbm, g_hbm, st_hbm, out_hbm,
             xbuf, gbuf, stv, acc):
    cid = lax.axis_index("c")
    sid = lax.axis_index("s")
    wid = sid * _NC + cid

    pltpu.sync_copy(st_hbm, stv)

    base = wid * _SEG_PER_TILE
    nj = _D // _L

    def _seg(s_local, _):
        s_idx = base + s_local
        a = stv[pl.ds(s_idx, _L)][0]
        e = stv[pl.ds(s_idx + 1, _L)][0]
        a8 = (a >> 3) << 3
        nch = (e - a8 + _R - 1) // _R

        def _chunk(c, regs):
            cs = pl.multiple_of(jnp.minimum(a8 + c * _R, _N - _R), 8)
            pltpu.sync_copy(x_hbm.at[pl.ds(cs, _R)], xbuf)
            pltpu.sync_copy(g_hbm.at[pl.ds(cs, _R)], gbuf.at[pl.ds(0, _R)])
            lo = jnp.maximum(a8 + c * _R, a)

            def _row(r, regs):
                g = cs + r
                valid = jnp.logical_and(g >= lo, g < e)
                wk = jnp.where(valid, gbuf[pl.ds(r, _L)][0], 0.0)
                return tuple(regs[j] + xbuf[r, pl.ds(j * _L, _L)] * wk
                             for j in range(nj))

            return lax.fori_loop(0, _R, _row, regs, unroll=8)

        zero_regs = tuple(jnp.zeros((_L,), jnp.float32) for _ in range(nj))
        regs = lax.fori_loop(0, nch, _chunk, zero_regs)
        for j in range(nj):
            acc[s_local, pl.ds(j * _L, _L)] = regs[j]
        return 0

    lax.fori_loop(0, _SEG_PER_TILE, _seg, 0)
    pltpu.sync_copy(acc, out_hbm.at[pl.ds(base, _SEG_PER_TILE)])


def kernel(x, batch, W, b):
    seg = batch.astype(jnp.int32)
    wr = W.reshape(1, _D).astype(jnp.float32)
    br = b.reshape(1, 1).astype(jnp.float32)

    starts, wgt = pl.pallas_call(
        _tc_fn,
        grid=(_G,),
        in_specs=[
            pl.BlockSpec((_BS, _D), lambda i: (i, 0)),
            pl.BlockSpec((1, _BS, 1), lambda i: (i, 0, 0)),
            pl.BlockSpec((1, _D), lambda i: (0, 0)),
            pl.BlockSpec((1, 1), lambda i: (0, 0)),
        ],
        out_specs=[
            pl.BlockSpec((1, _SP), lambda i: (0, 0)),
            pl.BlockSpec((1, _BS, 1), lambda i: (i, 0, 0)),
        ],
        out_shape=[
            jax.ShapeDtypeStruct((1, _SP), jnp.int32),
            jax.ShapeDtypeStruct((_G, _BS, 1), jnp.float32),
        ],
        compiler_params=pltpu.CompilerParams(
            dimension_semantics=("arbitrary",),
        ),
    )(x, seg.reshape(_G, _BS, 1), wr, br)

    mesh = plsc.VectorSubcoreMesh(
        core_axis_name="c", subcore_axis_name="s",
        num_cores=_NC, num_subcores=_NS)
    sc_fn = pl.kernel(
        _sc_body,
        out_type=jax.ShapeDtypeStruct((_S, _D), jnp.float32),
        mesh=mesh,
        scratch_types=[
            pltpu.VMEM((_R, _D), jnp.float32),
            pltpu.VMEM((_R + _L,), jnp.float32),
            pltpu.VMEM((_SP,), jnp.int32),
            pltpu.VMEM((_SEG_PER_TILE, _D), jnp.float32),
        ],
    )
    return sc_fn(x, wgt.reshape(_N), starts.reshape(_SP))
